# Initial kernel scaffold; baseline (speedup 1.0000x reference)
import functools

import jax
import jax.numpy as jnp
from jax import lax
from jax.experimental import pallas as pl
from jax.experimental.pallas import tpu as pltpu
from jax.experimental.pallas import tpu_sc as plsc

B = 4096
L = 300
DIM = 64
N = B * L

_info = plsc.get_sparse_core_info()
NC = _info.num_cores
NS = _info.num_subcores
NW = NC * NS

ROWS_PER_IDXROW = 128
R = 4
GROUP = R * ROWS_PER_IDXROW
IDX_ROWS = N // ROWS_PER_IDXROW
ROWS_PER_W = IDX_ROWS // NW
GROUPS_PER_W = ROWS_PER_W // R


def _gather_body(table_hbm, idx_hbm, out_hbm, idx_v, rows_v, sem):
    wid = lax.axis_index("s") * NC + lax.axis_index("c")
    row_base = wid * ROWS_PER_W

    def group(g, carry):
        row0 = row_base + g * R
        pltpu.sync_copy(idx_hbm.at[pl.ds(row0, R)], idx_v)
        copies = []
        for j in range(R):
            copies.append(
                pltpu.async_copy(
                    table_hbm.at[idx_v.at[j]],
                    rows_v.at[pl.ds(j * ROWS_PER_IDXROW, ROWS_PER_IDXROW)],
                    sem,
                )
            )
        for c in copies:
            c.wait()
        pltpu.sync_copy(
            rows_v, out_hbm.at[pl.ds(row0 * ROWS_PER_IDXROW, GROUP)]
        )
        return carry

    lax.fori_loop(0, GROUPS_PER_W, group, 0)


@functools.partial(
    pl.kernel,
    mesh=plsc.VectorSubcoreMesh(core_axis_name="c", subcore_axis_name="s"),
    out_type=jax.ShapeDtypeStruct((N, DIM), jnp.float32),
    scratch_types=[
        pltpu.VMEM((R, ROWS_PER_IDXROW), jnp.int32),
        pltpu.VMEM((GROUP, DIM), jnp.float32),
        pltpu.SemaphoreType.DMA,
    ],
)
def _embed_gather(table_hbm, idx_hbm, out_hbm, idx_v, rows_v, sem):
    _gather_body(table_hbm, idx_hbm, out_hbm, idx_v, rows_v, sem)


def kernel(input_ids, attention_mask, table):
    idx2d = input_ids.reshape(IDX_ROWS, ROWS_PER_IDXROW)
    flat = _embed_gather(table, idx2d)
    embeds = flat.reshape(B, L, DIM)
    return (embeds, embeds, attention_mask)


# speedup vs baseline: 21.5712x; 21.5712x over previous
import functools

import jax
import jax.numpy as jnp
from jax import lax
from jax.experimental import pallas as pl
from jax.experimental.pallas import tpu as pltpu
from jax.experimental.pallas import tpu_sc as plsc

B = 4096
L = 300
DIM = 64
N = B * L

_info = plsc.get_sparse_core_info()
NC = _info.num_cores
NS = _info.num_subcores
NW = NC * NS

ROWS_PER_IDXROW = 128
R = 4
GROUP = R * ROWS_PER_IDXROW
IDX_ROWS = N // ROWS_PER_IDXROW
ROWS_PER_W = IDX_ROWS // NW
GROUPS_PER_W = ROWS_PER_W // R


def _gather_body(table_hbm, idx_hbm, out_hbm, idx_v, rows_v, sem):
    wid = lax.axis_index("s") * NC + lax.axis_index("c")
    row_base = wid * ROWS_PER_W

    def group(g, carry):
        row0 = row_base + g * R
        pltpu.sync_copy(idx_hbm.at[pl.ds(row0, R)], idx_v)
        copies = []
        for j in range(R):
            copies.append(
                pltpu.async_copy(
                    table_hbm.at[idx_v.at[j]],
                    rows_v.at[pl.ds(j * ROWS_PER_IDXROW, ROWS_PER_IDXROW)],
                    sem,
                )
            )
        for c in copies:
            c.wait()
        pltpu.sync_copy(
            rows_v, out_hbm.at[pl.ds(row0 * ROWS_PER_IDXROW, GROUP)]
        )
        return carry

    lax.fori_loop(0, GROUPS_PER_W, group, 0)


@functools.partial(
    pl.kernel,
    mesh=plsc.VectorSubcoreMesh(core_axis_name="c", subcore_axis_name="s"),
    out_type=jax.ShapeDtypeStruct((N, DIM), jnp.float32),
    scratch_types=[
        pltpu.VMEM((R, ROWS_PER_IDXROW), jnp.int32),
        pltpu.VMEM((GROUP, DIM), jnp.float32),
        pltpu.SemaphoreType.DMA,
    ],
    compiler_params=pltpu.CompilerParams(use_tc_tiling_on_sc=False),
)
def _embed_gather(table_hbm, idx_hbm, out_hbm, idx_v, rows_v, sem):
    _gather_body(table_hbm, idx_hbm, out_hbm, idx_v, rows_v, sem)


def kernel(input_ids, attention_mask, table):
    idx2d = input_ids.reshape(IDX_ROWS, ROWS_PER_IDXROW)
    flat = _embed_gather(table, idx2d)
    embeds = flat.reshape(B, L, DIM)
    return (embeds, embeds, attention_mask)


# speedup vs baseline: 23.1170x; 1.0717x over previous
import functools

import jax
import jax.numpy as jnp
from jax import lax
from jax.experimental import pallas as pl
from jax.experimental.pallas import tpu as pltpu
from jax.experimental.pallas import tpu_sc as plsc

B = 4096
L = 300
DIM = 64
N = B * L

_info = plsc.get_sparse_core_info()
NC = _info.num_cores
NS = _info.num_subcores
NW = NC * NS

IDS_PER_ROW = 128
R = 4
GROUP = R * IDS_PER_ROW
IDX_ROWS = N // IDS_PER_ROW
ROWS_PER_W = IDX_ROWS // NW
GROUPS_PER_W = ROWS_PER_W // R


def _gather_body(table_hbm, idx_hbm, out_hbm, idx_v, rows0, rows1, sem0, sem1):
    wid = lax.axis_index("s") * NC + lax.axis_index("c")
    row_base = wid * ROWS_PER_W
    id_base = row_base * IDS_PER_ROW

    pltpu.sync_copy(idx_hbm.at[pl.ds(row_base, ROWS_PER_W)], idx_v)

    bufs = (rows0, rows1)
    sems = (sem0, sem1)

    def fire(g, p):
        for j in range(R):
            pltpu.async_copy(
                table_hbm.at[idx_v.at[g * R + j]],
                bufs[p].at[pl.ds(j * IDS_PER_ROW, IDS_PER_ROW)],
                sems[p],
            )

    def drain_store(g, p):
        pltpu.make_async_copy(
            out_hbm.at[pl.ds(id_base, GROUP)], bufs[p], sems[p]
        ).wait()
        pltpu.sync_copy(bufs[p], out_hbm.at[pl.ds(id_base + g * GROUP, GROUP)])

    fire(0, 0)

    def pair(k, carry):
        g_odd = 2 * k + 1
        fire(g_odd, 1)
        drain_store(g_odd - 1, 0)
        g_even = 2 * k + 2
        fire(g_even, 0)
        drain_store(g_even - 1, 1)
        return carry

    lax.fori_loop(0, (GROUPS_PER_W - 1) // 2, pair, 0)
    drain_store(GROUPS_PER_W - 1, 0)


@functools.partial(
    pl.kernel,
    mesh=plsc.VectorSubcoreMesh(core_axis_name="c", subcore_axis_name="s"),
    out_type=jax.ShapeDtypeStruct((N, DIM), jnp.float32),
    scratch_types=[
        pltpu.VMEM((ROWS_PER_W, IDS_PER_ROW), jnp.int32),
        pltpu.VMEM((GROUP, DIM), jnp.float32),
        pltpu.VMEM((GROUP, DIM), jnp.float32),
        pltpu.SemaphoreType.DMA,
        pltpu.SemaphoreType.DMA,
    ],
    compiler_params=pltpu.CompilerParams(use_tc_tiling_on_sc=False),
)
def _embed_gather(table_hbm, idx_hbm, out_hbm, idx_v, rows0, rows1, sem0, sem1):
    _gather_body(table_hbm, idx_hbm, out_hbm, idx_v, rows0, rows1, sem0, sem1)


def kernel(input_ids, attention_mask, table):
    idx2d = input_ids.reshape(IDX_ROWS, IDS_PER_ROW)
    flat = _embed_gather(table, idx2d)
    embeds = flat.reshape(B, L, DIM)
    return (embeds, embeds, attention_mask)
